# baseline (speedup 1.0000x reference)
import jax
import jax.numpy as jnp
from jax.experimental import pallas as pl


def _tk(f_ref, o_ref):
    c = f_ref.shape[1]
    o_ref[0, :, 0:c] = f_ref[0].T


def kernel(xyz, xyz_fp, features, features_fp, W, b):
    B, C, N = features.shape
    out = pl.pallas_call(
        _tk,
        grid=(B,),
        in_specs=[pl.BlockSpec((1, C, N), lambda i: (i, 0, 0))],
        out_specs=pl.BlockSpec((1, N, 2 * C), lambda i: (i, 0, 0)),
        out_shape=jax.ShapeDtypeStruct((B, N, 2 * C), features.dtype),
    )(features)
    return out[:, :, :C]
